# baseline (speedup 1.0000x reference)
import jax
import jax.numpy as jnp
from jax.experimental import pallas as pl
from jax.experimental.pallas import tpu as pltpu

BATCH = 64
SEQ = 1024
EMBED = 1024
TEXT = 768
EXP = 16
PAD = 128
TOPK = 2
S_CHUNK = 64
NSTEPS = SEQ // S_CHUNK


def _gating_kernel(x_ref, text_ref, w1_ref, b1_ref, w2_ref, b2_ref,
                   w_out, i_out, logits_out, acc_ref):
    step = pl.program_id(0)

    @pl.when(step == 0)
    def _init():
        acc_ref[...] = jnp.zeros_like(acc_ref)

    acc_ref[...] += jnp.sum(x_ref[...], axis=1)

    @pl.when(step == NSTEPS - 1)
    def _finish():
        x_mean = acc_ref[...] * (1.0 / SEQ)
        h = (
            jnp.dot(x_mean, w1_ref[:EMBED, :], preferred_element_type=jnp.float32)
            + jnp.dot(text_ref[...], w1_ref[EMBED:, :], preferred_element_type=jnp.float32)
            + b1_ref[...]
        )
        h = jnp.maximum(h, 0.0)
        logits = jnp.dot(h, w2_ref[...], preferred_element_type=jnp.float32) + b2_ref[...]
        logits_out[...] = logits

        col = jax.lax.broadcasted_iota(jnp.int32, (BATCH, PAD), 1)
        m1 = jnp.max(logits, axis=1, keepdims=True)
        i1 = jnp.min(jnp.where(logits == m1, col, PAD), axis=1, keepdims=True)
        masked = jnp.where(col == i1, -jnp.inf, logits)
        m2 = jnp.max(masked, axis=1, keepdims=True)
        i2 = jnp.min(jnp.where(masked == m2, col, PAD), axis=1, keepdims=True)

        e2 = jnp.exp(m2 - m1)
        denom = 1.0 + e2
        w_out[...] = jnp.where(col == 0, 1.0 / denom, jnp.where(col == 1, e2 / denom, 0.0))
        i_out[...] = jnp.where(col == 0, i1, jnp.where(col == 1, i2, 0))


def kernel(x, text_embedding, W1, b1, W2, b2):
    b1r = b1.reshape(1, EMBED)
    w2p = jnp.zeros((EMBED, PAD), W2.dtype).at[:, :EXP].set(W2)
    b2p = jnp.full((1, PAD), -1e30, b2.dtype).at[0, :EXP].set(b2)

    w, idx, logits = pl.pallas_call(
        _gating_kernel,
        grid=(NSTEPS,),
        in_specs=[
            pl.BlockSpec((BATCH, S_CHUNK, EMBED), lambda j: (0, j, 0)),
            pl.BlockSpec((BATCH, TEXT), lambda j: (0, 0)),
            pl.BlockSpec((EMBED + TEXT, EMBED), lambda j: (0, 0)),
            pl.BlockSpec((1, EMBED), lambda j: (0, 0)),
            pl.BlockSpec((EMBED, PAD), lambda j: (0, 0)),
            pl.BlockSpec((1, PAD), lambda j: (0, 0)),
        ],
        out_specs=[
            pl.BlockSpec((BATCH, PAD), lambda j: (0, 0)),
            pl.BlockSpec((BATCH, PAD), lambda j: (0, 0)),
            pl.BlockSpec((BATCH, PAD), lambda j: (0, 0)),
        ],
        out_shape=[
            jax.ShapeDtypeStruct((BATCH, PAD), jnp.float32),
            jax.ShapeDtypeStruct((BATCH, PAD), jnp.int32),
            jax.ShapeDtypeStruct((BATCH, PAD), jnp.float32),
        ],
        scratch_shapes=[pltpu.VMEM((BATCH, EMBED), jnp.float32)],
    )(x, text_embedding, W1, b1r, w2p, b2p)

    return w[:, :TOPK], idx[:, :TOPK], logits[:, :EXP]
